# baseline (speedup 1.0000x reference)
import functools

import jax
import jax.numpy as jnp
from jax import lax
from jax.experimental import pallas as pl
from jax.experimental.pallas import tpu as pltpu
from jax.experimental.pallas import tpu_sc as plsc

HIDDEN = 1024
NUM_ROWS = 2
BATCH = 4
SEQ_LEN = 8192
TOTAL = BATCH * SEQ_LEN

NC = 2
NS = 16
NW = NC * NS

LANES = 16
QROWS = 4
NPAT = 2 ** QROWS
PER_W = TOTAL // NW
QUADS_PER_W = PER_W // QROWS
QPS = 32
NSTEP = QUADS_PER_W // QPS
QBLK = QROWS * HIDDEN

_mesh = plsc.VectorSubcoreMesh(core_axis_name="c", subcore_axis_name="s")


@functools.partial(
    pl.kernel,
    mesh=_mesh,
    compiler_params=pltpu.CompilerParams(
        needs_layout_passes=False, use_tc_tiling_on_sc=True),
    out_type=jax.ShapeDtypeStruct((BATCH, SEQ_LEN, HIDDEN), jnp.float32),
    scratch_types=[
        pltpu.VMEM((PER_W,), jnp.int32),
        pltpu.VMEM((NUM_ROWS, HIDDEN), jnp.float32),
        pltpu.VMEM((NPAT, QROWS, HIDDEN), jnp.float32),
        pltpu.SemaphoreType.DMA,
    ],
)
def _sc_lookup(seg_hbm, table_hbm, out_hbm, idx_v, tbl_v, quad_v, sem):
    wid = lax.axis_index("s") * NC + lax.axis_index("c")
    row_base = wid * PER_W
    b = wid // (SEQ_LEN // PER_W)
    s_base = (wid % (SEQ_LEN // PER_W)) * PER_W
    pltpu.sync_copy(table_hbm, tbl_v)
    pltpu.sync_copy(
        seg_hbm.at[b, pl.ds(s_base, PER_W)], idx_v)
    @plsc.parallel_loop(0, NPAT * QBLK // LANES, unroll=8)
    def _build(n):
        p = n >> 8
        r = (n >> 6) & 3
        bit = (p >> (3 - r)) & 1
        col = (n & 63) * LANES
        val = tbl_v[bit, pl.ds(col, LANES)]
        quad_v[p, r, pl.ds(col, LANES)] = val

    def _issue_step(step):
        svecs = [idx_v[pl.ds(step * (QPS * QROWS) + m * LANES, LANES)]
                 for m in range((QPS * QROWS) // LANES)]
        descs = []
        for q in range(QPS):
            qid = None
            for r in range(QROWS):
                flat = q * QROWS + r
                s = svecs[flat // LANES][flat % LANES]
                qid = s if qid is None else qid * 2 + s
            s0 = s_base + step * (QPS * QROWS) + q * QROWS
            descs.append(pltpu.async_copy(
                quad_v.at[qid], out_hbm.at[b, pl.ds(s0, QROWS), :], sem))
        return descs

    head = _issue_step(0)

    def _body(step, _):
        descs = _issue_step(step)
        for d in descs:
            d.wait()
        return _

    lax.fori_loop(1, NSTEP, _body, None)
    for d in head:
        d.wait()


def kernel(segments, table):
    seg = segments.astype(jnp.int32)
    return _sc_lookup(seg, table)


# speedup vs baseline: 1.0185x; 1.0185x over previous
import functools

import jax
import jax.numpy as jnp
from jax import lax
from jax.experimental import pallas as pl
from jax.experimental.pallas import tpu as pltpu
from jax.experimental.pallas import tpu_sc as plsc

HIDDEN = 1024
NUM_ROWS = 2
BATCH = 4
SEQ_LEN = 8192
TOTAL = BATCH * SEQ_LEN

NC = 2
NS = 16
NW = NC * NS

LANES = 16
QROWS = 4
NPAT = 2 ** QROWS
PER_W = TOTAL // NW
QUADS_PER_W = PER_W // QROWS
QPS = 8
NSTEP = QUADS_PER_W // QPS
QBLK = QROWS * HIDDEN

_mesh = plsc.VectorSubcoreMesh(core_axis_name="c", subcore_axis_name="s")


@functools.partial(
    pl.kernel,
    mesh=_mesh,
    compiler_params=pltpu.CompilerParams(
        needs_layout_passes=False, use_tc_tiling_on_sc=True),
    out_type=jax.ShapeDtypeStruct((BATCH, SEQ_LEN, HIDDEN), jnp.float32),
    scratch_types=[
        pltpu.VMEM((PER_W,), jnp.int32),
        pltpu.VMEM((NUM_ROWS, HIDDEN), jnp.float32),
        pltpu.VMEM((NPAT, QROWS, HIDDEN), jnp.float32),
        pltpu.SemaphoreType.DMA,
    ],
)
def _sc_lookup(seg_hbm, table_hbm, out_hbm, idx_v, tbl_v, quad_v, sem):
    wid = lax.axis_index("s") * NC + lax.axis_index("c")
    row_base = wid * PER_W
    b = wid // (SEQ_LEN // PER_W)
    s_base = (wid % (SEQ_LEN // PER_W)) * PER_W
    pltpu.sync_copy(table_hbm, tbl_v)
    pltpu.sync_copy(
        seg_hbm.at[b, pl.ds(s_base, PER_W)], idx_v)
    @plsc.parallel_loop(0, NPAT * QBLK // LANES, unroll=8)
    def _build(n):
        p = n >> 8
        r = (n >> 6) & 3
        bit = (p >> (3 - r)) & 1
        col = (n & 63) * LANES
        val = tbl_v[bit, pl.ds(col, LANES)]
        quad_v[p, r, pl.ds(col, LANES)] = val

    def _issue_step(step):
        svecs = [idx_v[pl.ds(step * (QPS * QROWS) + m * LANES, LANES)]
                 for m in range((QPS * QROWS) // LANES)]
        descs = []
        for q in range(QPS):
            qid = None
            for r in range(QROWS):
                flat = q * QROWS + r
                s = svecs[flat // LANES][flat % LANES]
                qid = s if qid is None else qid * 2 + s
            s0 = s_base + step * (QPS * QROWS) + q * QROWS
            descs.append(pltpu.async_copy(
                quad_v.at[qid], out_hbm.at[b, pl.ds(s0, QROWS), :], sem))
        return descs

    head = _issue_step(0)

    def _body(step, _):
        descs = _issue_step(step)
        for d in descs:
            d.wait()
        return _

    lax.fori_loop(1, NSTEP, _body, None)
    for d in head:
        d.wait()


def kernel(segments, table):
    seg = segments.astype(jnp.int32)
    return _sc_lookup(seg, table)
